# baseline (speedup 1.0000x reference)
import functools

import jax
import jax.numpy as jnp
from jax import lax
from jax.experimental import pallas as pl
from jax.experimental.pallas import tpu as pltpu
from jax.experimental.pallas import tpu_sc as plsc

T = 16384
D = 4096
L = 16
NC = 2
NS = 16
NW = NC * NS
ROWS_PER_W = T // NW
C = 4
NCHUNK = ROWS_PER_W // C
NBUF = 6
AHEAD = 2
DL = D // L


def _bf16_rtne(v):
    c = v * jnp.float32(65537.0)
    return c - (c - v)


def _body(x_hbm, wgt_hbm, x0_hbm, x1_hbm, xo_hbm,
          w01_v, wd_v, zc_v, xcs, sem_rs, sem_ws, sem_z):
    cid = lax.axis_index("c")
    sid = lax.axis_index("s")
    wid = sid * NC + cid
    base = wid * ROWS_PER_W

    pltpu.sync_copy(wgt_hbm, w01_v)

    def _init(j, carry):
        sl = pl.ds(j * L, L)
        wd_v[sl] = _bf16_rtne(w01_v[0, sl]) - _bf16_rtne(w01_v[1, sl])
        zc_v[sl] = jnp.zeros((L,), jnp.float32)
        return carry
    lax.fori_loop(0, DL, _init, 0, unroll=8)

    def _read(k, b):
        pltpu.async_copy(x_hbm.at[pl.ds(base + k * C, C)], xcs[b], sem_rs[b])

    def _wait_read(b):
        pltpu.make_async_copy(x_hbm.at[pl.ds(base, C)], xcs[b],
                              sem_rs[b]).wait()

    def _drain_writes(b):
        for _ in range(C):
            pltpu.make_async_copy(xcs[b].at[0], x0_hbm.at[base],
                                  sem_ws[b]).wait()
        pltpu.make_async_copy(xcs[b], xo_hbm.at[pl.ds(base, C)],
                              sem_ws[b]).wait()

    def _process(k, b):
        xc_v = xcs[b]
        _wait_read(b)
        def _dot(j, accs):
            sl = pl.ds(j * L, L)
            w = wd_v[sl]
            return tuple(accs[i] + _bf16_rtne(xc_v[i, sl]) * w
                         for i in range(C))
        accs = lax.fori_loop(
            0, DL, _dot, tuple(jnp.zeros((L,), jnp.float32)
                               for _ in range(C)), unroll=4)
        ds_ = []
        gvs = []
        for i in range(C):
            parts = [accs[i][l] for l in range(L)]
            while len(parts) > 1:
                parts = [parts[p] + parts[p + 1]
                         for p in range(0, len(parts), 2)]
            d = parts[0]
            dv = jnp.full((L,), d, jnp.float32)
            gvs.append(1.0 / (1.0 + jnp.exp(-jnp.abs(dv))))
            ds_.append(d)

        def _scale(j, carry):
            sl = pl.ds(j * L, L)
            for i in range(C):
                xc_v[i, sl] = xc_v[i, sl] * gvs[i]
            return carry
        lax.fori_loop(0, DL, _scale, 0, unroll=2)

        r0 = base + k * C
        for i in range(C):
            row = r0 + i

            @pl.when(ds_[i] >= 0.0)
            def _():
                pltpu.async_copy(xc_v.at[i], x0_hbm.at[row], sem_ws[b])
                pltpu.async_copy(zc_v, x1_hbm.at[row], sem_z)

            @pl.when(ds_[i] < 0.0)
            def _():
                pltpu.async_copy(zc_v, x0_hbm.at[row], sem_z)
                pltpu.async_copy(xc_v.at[i], x1_hbm.at[row], sem_ws[b])
        pltpu.async_copy(xc_v, xo_hbm.at[pl.ds(r0, C)], sem_ws[b])

    for b in range(NBUF):
        _read(b, b)

    def _step(k3, carry):
        k = k3 * NBUF
        for b in range(NBUF):
            q = k + b
            ahead = q + AHEAD
            tb = (b + AHEAD) % NBUF

            @pl.when(jnp.logical_and(ahead >= NBUF, ahead < NCHUNK))
            def _():
                _drain_writes(tb)
                _read(ahead, tb)
            _process(q, b)
        return carry
    lax.fori_loop(0, NCHUNK // NBUF, _step, 0)

    for b in range(NCHUNK % NBUF):
        _process((NCHUNK // NBUF) * NBUF + b, b)

    for b in range(NBUF):
        _drain_writes(b)
    def _drain_z(_, carry):
        pltpu.make_async_copy(zc_v, x0_hbm.at[base], sem_z).wait()
        return carry
    lax.fori_loop(0, ROWS_PER_W, _drain_z, 0)


@jax.jit
def _run(x, wgt):
    mesh = plsc.VectorSubcoreMesh(core_axis_name="c", subcore_axis_name="s")
    f = functools.partial(
        pl.kernel,
        mesh=mesh,
        out_type=[
            jax.ShapeDtypeStruct((T, D), jnp.float32),
            jax.ShapeDtypeStruct((T, D), jnp.float32),
            jax.ShapeDtypeStruct((T, D), jnp.float32),
        ],
        scratch_types=[
            pltpu.VMEM((2, D), jnp.float32),
            pltpu.VMEM((D,), jnp.float32),
            pltpu.VMEM((D,), jnp.float32),
            [pltpu.VMEM((C, D), jnp.float32) for _ in range(NBUF)],
            [pltpu.SemaphoreType.DMA for _ in range(NBUF)],
            [pltpu.SemaphoreType.DMA for _ in range(NBUF)],
            pltpu.SemaphoreType.DMA,
        ],
    )(_body)
    return f(x, wgt)


def kernel(x, Wg):
    wgt = Wg.T
    x0, x1, xo = _run(x, wgt)
    return (x0, x1, xo)


# speedup vs baseline: 1.3407x; 1.3407x over previous
import functools

import jax
import jax.numpy as jnp
from jax import lax
from jax.experimental import pallas as pl
from jax.experimental.pallas import tpu as pltpu
from jax.experimental.pallas import tpu_sc as plsc

T = 16384
D = 4096
L = 16
NC = 2
NS = 16
NW = NC * NS
ROWS_PER_W = T // NW
C = 4
NCHUNK = ROWS_PER_W // C
NBUF = 6
AHEAD = 2
DL = D // L

TC_R = 256


def _bf16_rtne(v):
    c = v * jnp.float32(65537.0)
    return c - (c - v)


def _sc_body(x_hbm, wgt_hbm, x0_hbm, w01_v, wd_v, zc_v, xcs, sem_rs, sem_ws):
    cid = lax.axis_index("c")
    sid = lax.axis_index("s")
    wid = sid * NC + cid
    base = wid * ROWS_PER_W

    pltpu.sync_copy(wgt_hbm, w01_v)

    def _init(j, carry):
        sl = pl.ds(j * L, L)
        wd_v[sl] = _bf16_rtne(w01_v[0, sl]) - _bf16_rtne(w01_v[1, sl])
        zc_v[sl] = jnp.zeros((L,), jnp.float32)
        return carry
    lax.fori_loop(0, DL, _init, 0, unroll=8)

    def _read(k, b):
        pltpu.async_copy(x_hbm.at[pl.ds(base + k * C, C)], xcs[b], sem_rs[b])

    def _wait_read(b):
        pltpu.make_async_copy(x_hbm.at[pl.ds(base, C)], xcs[b],
                              sem_rs[b]).wait()

    def _drain_writes(b):
        for _ in range(C):
            pltpu.make_async_copy(xcs[b].at[0], x0_hbm.at[base],
                                  sem_ws[b]).wait()

    def _process(k, b):
        xc_v = xcs[b]
        _wait_read(b)
        def _dot(j, accs):
            sl = pl.ds(j * L, L)
            w = wd_v[sl]
            return tuple(accs[i] + _bf16_rtne(xc_v[i, sl]) * w
                         for i in range(C))
        accs = lax.fori_loop(
            0, DL, _dot, tuple(jnp.zeros((L,), jnp.float32)
                               for _ in range(C)), unroll=4)
        r0 = base + k * C
        for i in range(C):
            parts = [accs[i][l] for l in range(L)]
            while len(parts) > 1:
                parts = [parts[p] + parts[p + 1]
                         for p in range(0, len(parts), 2)]
            d = parts[0]
            row = r0 + i

            @pl.when(d >= 0.0)
            def _():
                dv = jnp.full((L,), d, jnp.float32)
                gv = 1.0 / (1.0 + jnp.exp(-dv))

                def _scale(j, carry2):
                    sl = pl.ds(j * L, L)
                    xc_v[i, sl] = xc_v[i, sl] * gv
                    return carry2
                lax.fori_loop(0, DL, _scale, 0, unroll=4)
                pltpu.async_copy(xc_v.at[i], x0_hbm.at[row], sem_ws[b])

            @pl.when(d < 0.0)
            def _():
                pltpu.async_copy(zc_v, x0_hbm.at[row], sem_ws[b])

    for b in range(NBUF):
        _read(b, b)

    def _step(k3, carry):
        k = k3 * NBUF
        for b in range(NBUF):
            q = k + b
            ahead = q + AHEAD
            tb = (b + AHEAD) % NBUF

            @pl.when(jnp.logical_and(ahead >= NBUF, ahead < NCHUNK))
            def _():
                _drain_writes(tb)
                _read(ahead, tb)
            _process(q, b)
        return carry
    lax.fori_loop(0, NCHUNK // NBUF, _step, 0)

    for b in range(NCHUNK % NBUF):
        _process((NCHUNK // NBUF) * NBUF + b, b)

    for b in range(NBUF):
        _drain_writes(b)


def _tc_body(x_ref, wg_ref, xo_ref, x1_ref):
    xb = x_ref[...]
    logits = lax.dot_general(xb, wg_ref[...], (((1,), (0,)), ((), ())),
                             preferred_element_type=jnp.float32)
    d = logits[:, 0:1] - logits[:, 1:2]
    gate = 1.0 / (1.0 + jnp.exp(-jnp.abs(d)))
    xg = xb * gate
    xo_ref[...] = xg
    x1_ref[...] = jnp.where(d < 0.0, xg, jnp.float32(0.0))


def _tc_call(x, wg):
    grid = (T // TC_R,)
    return pl.pallas_call(
        _tc_body,
        grid=grid,
        in_specs=[
            pl.BlockSpec((TC_R, D), lambda i: (i, 0)),
            pl.BlockSpec((D, 2), lambda i: (0, 0)),
        ],
        out_specs=[
            pl.BlockSpec((TC_R, D), lambda i: (i, 0)),
            pl.BlockSpec((TC_R, D), lambda i: (i, 0)),
        ],
        out_shape=[
            jax.ShapeDtypeStruct((T, D), jnp.float32),
            jax.ShapeDtypeStruct((T, D), jnp.float32),
        ],
    )(x, wg)


@jax.jit
def _run(x, wg, wgt):
    mesh = plsc.VectorSubcoreMesh(core_axis_name="c", subcore_axis_name="s")
    sc = functools.partial(
        pl.kernel,
        mesh=mesh,
        out_type=jax.ShapeDtypeStruct((T, D), jnp.float32),
        scratch_types=[
            pltpu.VMEM((2, D), jnp.float32),
            pltpu.VMEM((D,), jnp.float32),
            pltpu.VMEM((D,), jnp.float32),
            [pltpu.VMEM((C, D), jnp.float32) for _ in range(NBUF)],
            [pltpu.SemaphoreType.DMA for _ in range(NBUF)],
            [pltpu.SemaphoreType.DMA for _ in range(NBUF)],
        ],
    )(_sc_body)
    x0 = sc(x, wgt)
    xo, x1 = _tc_call(x, wg)
    return x0, x1, xo


def kernel(x, Wg):
    wgt = Wg.T
    x0, x1, xo = _run(x, Wg, wgt)
    return (x0, x1, xo)
